# Initial kernel scaffold; baseline (speedup 1.0000x reference)
import jax
import jax.numpy as jnp
from jax import lax
from jax.experimental import pallas as pl
from jax.experimental.pallas import tpu as pltpu
from jax.experimental.pallas import tpu_sc as plsc

N_NODES = 10000
N_EDGES = 320000
HID = 128
BS = 10
N_PAD = 10240


def _node_enc_body(x_ref, qa_ref, nt_ref, ns_ref, wnt_ref, bnt_ref, wx_ref,
                   bx_ref, wg_ref, asv_ref, adv_ref,
                   xl_ref, asrc_ref, adst_ref, amax_ref):
    n = x_ref.shape[0]
    bs = qa_ref.shape[0]
    npb = n // bs
    ri = lax.broadcasted_iota(jnp.int32, (n, bs), 0)
    ci = lax.broadcasted_iota(jnp.int32, (n, bs), 1)
    ohq = (ri == ci * npb).astype(jnp.float32)
    xq = jnp.dot(ohq, qa_ref[...], preferred_element_type=jnp.float32)
    ri1 = lax.broadcasted_iota(jnp.int32, (n, 1), 0)
    x = jnp.where((ri1 % npb) == 0, xq, x_ref[...])
    extras = (jnp.dot(jnp.concatenate([nt_ref[...], ns_ref[...]], axis=1),
                      wnt_ref[...], preferred_element_type=jnp.float32)
              + bnt_ref[...])
    xx = jax.nn.relu(jnp.concatenate([x, extras], axis=1))
    h = jax.nn.relu(jnp.dot(xx, wx_ref[...],
                            preferred_element_type=jnp.float32) + bx_ref[...])
    xl = jnp.dot(h, wg_ref[...], preferred_element_type=jnp.float32)
    xl_ref[...] = xl
    a_s = jnp.dot(xl, asv_ref[...], preferred_element_type=jnp.float32)
    a_d = jnp.dot(xl, adv_ref[...], preferred_element_type=jnp.float32)
    asrc_ref[...] = a_s
    adst_ref[...] = a_d
    amax_ref[...] = jnp.concatenate(
        [jnp.max(a_s).reshape(1, 1), jnp.max(a_d).reshape(1, 1)], axis=1)


def _node_enc(x, qa, nt, ns, wnt, bnt, wx, bx, wg, asv, adv):
    n = x.shape[0]
    return pl.pallas_call(
        _node_enc_body,
        out_shape=[
            jax.ShapeDtypeStruct((n, HID), jnp.float32),
            jax.ShapeDtypeStruct((n, 1), jnp.float32),
            jax.ShapeDtypeStruct((n, 1), jnp.float32),
            jax.ShapeDtypeStruct((1, 2), jnp.float32),
        ],
    )(x, qa, nt, ns, wnt, bnt, wx, bx, wg, asv, adv)


_EDGE_BLK = 2000


def _edge_enc_body(ea_ref, w1_ref, b1_ref, w2_ref, b2_ref, we_ref, aev_ref,
                   ae_ref, aemax_ref):
    t = jax.nn.relu(jnp.dot(ea_ref[...], w1_ref[...],
                            preferred_element_type=jnp.float32) + b1_ref[...])
    u = jax.nn.relu(jnp.dot(t, w2_ref[...],
                            preferred_element_type=jnp.float32) + b2_ref[...])
    ve = jnp.dot(we_ref[...], aev_ref[...],
                 preferred_element_type=jnp.float32)
    ae = jnp.dot(u, ve, preferred_element_type=jnp.float32)
    ae_ref[...] = ae
    m = jnp.max(ae).reshape(1, 1)
    i = pl.program_id(0)

    @pl.when(i == 0)
    def _():
        aemax_ref[...] = m

    @pl.when(i > 0)
    def _():
        aemax_ref[...] = jnp.maximum(aemax_ref[...], m)


def _edge_enc(ea, w1, b1, w2, b2, we, aev):
    e, ein = ea.shape
    grid = e // _EDGE_BLK
    return pl.pallas_call(
        _edge_enc_body,
        grid=(grid,),
        in_specs=[
            pl.BlockSpec((_EDGE_BLK, ein), lambda i: (i, 0)),
            pl.BlockSpec((ein, HID), lambda i: (0, 0)),
            pl.BlockSpec((1, HID), lambda i: (0, 0)),
            pl.BlockSpec((HID, HID), lambda i: (0, 0)),
            pl.BlockSpec((1, HID), lambda i: (0, 0)),
            pl.BlockSpec((HID, HID), lambda i: (0, 0)),
            pl.BlockSpec((HID, 1), lambda i: (0, 0)),
        ],
        out_specs=[
            pl.BlockSpec((_EDGE_BLK, 1), lambda i: (i, 0)),
            pl.BlockSpec((1, 1), lambda i: (0, 0)),
        ],
        out_shape=[
            jax.ShapeDtypeStruct((e, 1), jnp.float32),
            jax.ShapeDtypeStruct((1, 1), jnp.float32),
        ],
    )(ea, w1, b1, w2, b2, we, aev)


_P1_CHUNK = 400
_P1_EPT = N_EDGES // 16
_P2_CHUNK = 80
_P2_EPT = N_EDGES // 32
_ROWS_PT = N_PAD // 16


def _mp_body(src_hbm, dst_hbm, ae_hbm, asrc_hbm, adst_hbm, mh_hbm, xl_hbm,
             out_hbm,
             asrc_v, adst_v, den_v, red_v, dtot_v, rows_v, sidx_v, didx_v,
             aec_v, a_v, mh_v, p1s_v, p1d_v, p1a_v,
             sh_hout, sh_den, sh_dentot, sem):
    cid = lax.axis_index("c")
    sid = lax.axis_index("s")

    pltpu.sync_copy(asrc_hbm, asrc_v)
    pltpu.sync_copy(adst_hbm, adst_v)
    pltpu.sync_copy(mh_hbm, mh_v)
    mh = mh_v[...]
    z16 = jnp.zeros((16,), jnp.float32)

    def zden(i, c):
        den_v[pl.ds(i * 16, 16)] = z16
        return c
    lax.fori_loop(0, N_PAD // 16, zden, 0)

    def zrows(i, c):
        for k in range(HID // 16):
            rows_v[i, pl.ds(k * 16, 16)] = z16
        return c
    lax.fori_loop(0, _P2_CHUNK, zrows, 0)
    for b in range(_ROWS_PT // _P2_CHUNK):
        pltpu.sync_copy(
            rows_v, sh_hout.at[pl.ds(sid * _ROWS_PT + b * _P2_CHUNK,
                                     _P2_CHUNK)])
    plsc.subcore_barrier()

    def p1_chunk(c, carry):
        base = sid * _P1_EPT + c * _P1_CHUNK
        pltpu.sync_copy(src_hbm.at[pl.ds(base, _P1_CHUNK)], p1s_v)
        pltpu.sync_copy(dst_hbm.at[pl.ds(base, _P1_CHUNK)], p1d_v)
        pltpu.sync_copy(ae_hbm.at[pl.ds(base, _P1_CHUNK)], p1a_v)

        def inner(j, cc):
            s16 = p1s_v[pl.ds(j * 16, 16)]
            d16 = p1d_v[pl.ds(j * 16, 16)]
            ae16 = p1a_v[pl.ds(j * 16, 16)]
            al = (plsc.load_gather(asrc_v, [s16])
                  + plsc.load_gather(adst_v, [d16]) + ae16)
            al = jnp.maximum(al, 0.2 * al)
            ex = jnp.exp(al - mh)
            plsc.addupdate_scatter(den_v, [d16], ex)
            return cc
        lax.fori_loop(0, _P1_CHUNK // 16, inner, 0)
        return carry
    lax.fori_loop(0, _P1_EPT // _P1_CHUNK, p1_chunk, 0)

    pltpu.sync_copy(den_v, sh_den.at[sid])
    plsc.subcore_barrier()
    pltpu.sync_copy(sh_den.at[:, pl.ds(sid * _ROWS_PT, _ROWS_PT)], red_v)

    def red_grp(g, c):
        acc = red_v[0, pl.ds(g * 16, 16)]
        for t in range(1, 16):
            acc = acc + red_v[t, pl.ds(g * 16, 16)]
        dtot_v[pl.ds(g * 16, 16)] = acc
        return c
    lax.fori_loop(0, _ROWS_PT // 16, red_grp, 0)
    pltpu.sync_copy(dtot_v, sh_dentot.at[pl.ds(sid * _ROWS_PT, _ROWS_PT)])
    plsc.subcore_barrier()
    pltpu.sync_copy(sh_dentot, den_v)

    def p2_chunk(c, carry):
        base = (cid * 16 + sid) * _P2_EPT + c * _P2_CHUNK
        pltpu.sync_copy(src_hbm.at[pl.ds(base, _P2_CHUNK)], sidx_v)
        pltpu.sync_copy(dst_hbm.at[pl.ds(base, _P2_CHUNK)], didx_v)
        pltpu.sync_copy(ae_hbm.at[pl.ds(base, _P2_CHUNK)], aec_v)
        pltpu.async_copy(xl_hbm.at[sidx_v], rows_v, sem).wait()

        def attn(j, cc):
            s16 = sidx_v[pl.ds(j * 16, 16)]
            d16 = didx_v[pl.ds(j * 16, 16)]
            ae16 = aec_v[pl.ds(j * 16, 16)]
            al = (plsc.load_gather(asrc_v, [s16])
                  + plsc.load_gather(adst_v, [d16]) + ae16)
            al = jnp.maximum(al, 0.2 * al)
            ex = jnp.exp(al - mh)
            dn = plsc.load_gather(den_v, [d16])
            a_v[pl.ds(j * 16, 16)] = ex / (dn + 1e-16)
            return cc
        lax.fori_loop(0, _P2_CHUNK // 16, attn, 0)

        def scale(e, cc):
            ab = plsc.load_gather(a_v, [jnp.full((16,), e, jnp.int32)])
            for k in range(HID // 16):
                rows_v[e, pl.ds(k * 16, 16)] = (
                    rows_v[e, pl.ds(k * 16, 16)] * ab)
            return cc
        lax.fori_loop(0, _P2_CHUNK, scale, 0)
        pltpu.sync_copy(rows_v, sh_hout.at[didx_v], add=True)
        return carry
    lax.fori_loop(0, _P2_EPT // _P2_CHUNK, p2_chunk, 0)
    plsc.subcore_barrier()
    pltpu.sync_copy(sh_hout.at[pl.ds(sid * _ROWS_PT, _ROWS_PT)],
                    out_hbm.at[cid, pl.ds(sid * _ROWS_PT, _ROWS_PT)])


def _message_passing(src, dst, ae, asrc, adst, mh16, xl):
    mesh = plsc.VectorSubcoreMesh(core_axis_name="c", subcore_axis_name="s")
    f32, i32 = jnp.float32, jnp.int32
    k = pl.kernel(
        _mp_body,
        out_type=jax.ShapeDtypeStruct((2, N_PAD, HID), f32),
        mesh=mesh,
        scratch_types=[
            pltpu.VMEM((N_NODES,), f32),
            pltpu.VMEM((N_NODES,), f32),
            pltpu.VMEM((N_PAD,), f32),
            pltpu.VMEM((16, _ROWS_PT), f32),
            pltpu.VMEM((_ROWS_PT,), f32),
            pltpu.VMEM((_P2_CHUNK, HID), f32),
            pltpu.VMEM((_P2_CHUNK,), i32),
            pltpu.VMEM((_P2_CHUNK,), i32),
            pltpu.VMEM((_P2_CHUNK,), f32),
            pltpu.VMEM((_P2_CHUNK,), f32),
            pltpu.VMEM((16,), f32),
            pltpu.VMEM((_P1_CHUNK,), i32),
            pltpu.VMEM((_P1_CHUNK,), i32),
            pltpu.VMEM((_P1_CHUNK,), f32),
            pltpu.VMEM_SHARED((N_PAD, HID), f32),
            pltpu.VMEM_SHARED((16, N_PAD), f32),
            pltpu.VMEM_SHARED((N_PAD,), f32),
            pltpu.SemaphoreType.DMA,
        ],
    )
    return k(src, dst, ae, asrc, adst, mh16, xl)


def _finish_body(p0_ref, p1_ref, bg_ref, n2g_ref, h0_ref, p_ref):
    hout = p0_ref[...] + p1_ref[...] + bg_ref[...]
    n = hout.shape[0]
    bs = p_ref.shape[0]
    gi = lax.broadcasted_iota(jnp.int32, (n, bs), 1)
    oh = (n2g_ref[...] == gi).astype(jnp.float32)
    dn = (((0,), (0,)), ((), ()))
    psum = lax.dot_general(oh, hout, dn, preferred_element_type=jnp.float32)
    cnt = lax.dot_general(oh, jnp.ones((n, 1), jnp.float32), dn,
                          preferred_element_type=jnp.float32)
    p_ref[...] = jax.nn.relu(psum / jnp.maximum(cnt, 1.0))
    ri = lax.broadcasted_iota(jnp.int32, (n, bs), 0)
    oh0 = (ri == gi * (n // bs)).astype(jnp.float32)
    h0_ref[...] = lax.dot_general(oh0, hout, dn,
                                  preferred_element_type=jnp.float32)


def _finish(part0, part1, bg, n2g):
    return pl.pallas_call(
        _finish_body,
        out_shape=[
            jax.ShapeDtypeStruct((BS, HID), jnp.float32),
            jax.ShapeDtypeStruct((BS, HID), jnp.float32),
        ],
    )(part0, part1, bg, n2g)


def kernel(qa_emb, x, node_ids, node_types, node_scores, edge_index,
           edge_type, edge_attr, node2graph, W_nt, b_nt, W_x2h, b_x2h, W_e1,
           b_e1, W_e2, b_e2, W_gat, att_src, att_dst, W_edge, att_edge,
           b_gat):
    xl, a_s, a_d, amax = _node_enc(
        x, qa_emb, node_types, node_scores, W_nt, b_nt.reshape(1, -1),
        W_x2h, b_x2h.reshape(1, -1), W_gat,
        att_src.reshape(-1, 1), att_dst.reshape(-1, 1))
    ae, aemax = _edge_enc(edge_attr, W_e1, b_e1.reshape(1, -1), W_e2,
                          b_e2.reshape(1, -1), W_edge,
                          att_edge.reshape(-1, 1))
    msum = amax[0, 0] + amax[0, 1] + aemax[0, 0]
    mh = jnp.maximum(msum, 0.2 * msum)
    mh16 = jnp.broadcast_to(mh, (16,))
    hpart = _message_passing(edge_index[0], edge_index[1], ae.reshape(-1),
                             a_s.reshape(-1), a_d.reshape(-1), mh16, xl)
    h0, p = _finish(hpart[0, :N_NODES], hpart[1, :N_NODES],
                    b_gat.reshape(1, -1), node2graph.reshape(-1, 1))
    return (h0, p)


# speedup vs baseline: 7.5025x; 7.5025x over previous
import jax
import jax.numpy as jnp
from jax import lax
from jax.experimental import pallas as pl
from jax.experimental.pallas import tpu as pltpu
from jax.experimental.pallas import tpu_sc as plsc

N_NODES = 10000
N_EDGES = 320000
HID = 128
BS = 10
N_PAD = 10240


def _node_enc_body(x_ref, qa_ref, nt_ref, ns_ref, wnt_ref, bnt_ref, wx_ref,
                   bx_ref, wg_ref, asv_ref, adv_ref,
                   xl_ref, asrc_ref, adst_ref, amax_ref):
    n = x_ref.shape[0]
    bs = qa_ref.shape[0]
    npb = n // bs
    ri = lax.broadcasted_iota(jnp.int32, (n, bs), 0)
    ci = lax.broadcasted_iota(jnp.int32, (n, bs), 1)
    ohq = (ri == ci * npb).astype(jnp.float32)
    xq = jnp.dot(ohq, qa_ref[...], preferred_element_type=jnp.float32)
    ri1 = lax.broadcasted_iota(jnp.int32, (n, 1), 0)
    x = jnp.where((ri1 % npb) == 0, xq, x_ref[...])
    extras = (jnp.dot(jnp.concatenate([nt_ref[...], ns_ref[...]], axis=1),
                      wnt_ref[...], preferred_element_type=jnp.float32)
              + bnt_ref[...])
    xx = jax.nn.relu(jnp.concatenate([x, extras], axis=1))
    h = jax.nn.relu(jnp.dot(xx, wx_ref[...],
                            preferred_element_type=jnp.float32) + bx_ref[...])
    xl = jnp.dot(h, wg_ref[...], preferred_element_type=jnp.float32)
    xl_ref[...] = xl
    a_s = jnp.dot(xl, asv_ref[...], preferred_element_type=jnp.float32)
    a_d = jnp.dot(xl, adv_ref[...], preferred_element_type=jnp.float32)
    asrc_ref[...] = a_s
    adst_ref[...] = a_d
    amax_ref[...] = jnp.concatenate(
        [jnp.max(a_s).reshape(1, 1), jnp.max(a_d).reshape(1, 1)], axis=1)


def _node_enc(x, qa, nt, ns, wnt, bnt, wx, bx, wg, asv, adv):
    n = x.shape[0]
    return pl.pallas_call(
        _node_enc_body,
        out_shape=[
            jax.ShapeDtypeStruct((n, HID), jnp.float32),
            jax.ShapeDtypeStruct((n, 1), jnp.float32),
            jax.ShapeDtypeStruct((n, 1), jnp.float32),
            jax.ShapeDtypeStruct((1, 2), jnp.float32),
        ],
    )(x, qa, nt, ns, wnt, bnt, wx, bx, wg, asv, adv)


_EDGE_BLK = 2000


def _edge_enc_body(ea_ref, w1_ref, b1_ref, w2_ref, b2_ref, we_ref, aev_ref,
                   ae_ref, aemax_ref):
    t = jax.nn.relu(jnp.dot(ea_ref[...], w1_ref[...],
                            preferred_element_type=jnp.float32) + b1_ref[...])
    u = jax.nn.relu(jnp.dot(t, w2_ref[...],
                            preferred_element_type=jnp.float32) + b2_ref[...])
    ve = jnp.dot(we_ref[...], aev_ref[...],
                 preferred_element_type=jnp.float32)
    ae = jnp.dot(u, ve, preferred_element_type=jnp.float32)
    ae_ref[...] = ae
    m = jnp.max(ae).reshape(1, 1)
    i = pl.program_id(0)

    @pl.when(i == 0)
    def _():
        aemax_ref[...] = m

    @pl.when(i > 0)
    def _():
        aemax_ref[...] = jnp.maximum(aemax_ref[...], m)


def _edge_enc(ea, w1, b1, w2, b2, we, aev):
    e, ein = ea.shape
    grid = e // _EDGE_BLK
    return pl.pallas_call(
        _edge_enc_body,
        grid=(grid,),
        in_specs=[
            pl.BlockSpec((_EDGE_BLK, ein), lambda i: (i, 0)),
            pl.BlockSpec((ein, HID), lambda i: (0, 0)),
            pl.BlockSpec((1, HID), lambda i: (0, 0)),
            pl.BlockSpec((HID, HID), lambda i: (0, 0)),
            pl.BlockSpec((1, HID), lambda i: (0, 0)),
            pl.BlockSpec((HID, HID), lambda i: (0, 0)),
            pl.BlockSpec((HID, 1), lambda i: (0, 0)),
        ],
        out_specs=[
            pl.BlockSpec((_EDGE_BLK, 1), lambda i: (i, 0)),
            pl.BlockSpec((1, 1), lambda i: (0, 0)),
        ],
        out_shape=[
            jax.ShapeDtypeStruct((e, 1), jnp.float32),
            jax.ShapeDtypeStruct((1, 1), jnp.float32),
        ],
    )(ea, w1, b1, w2, b2, we, aev)


_P1_CHUNK = 80
_P1_EPT = N_EDGES // 16
_P2_CHUNK = 80
_P2_EPT = N_EDGES // 32
_ROWS_PT = N_PAD // 16


def _mp_body(src_hbm, dst_hbm, ae_hbm, asrc_hbm, adst_hbm, mh_hbm, xl_hbm,
             out_hbm,
             asrc_v, adst_v, den_v, rows_v, sidx_v, didx_v,
             aec_v, a_v, mh_v, p1s_v, p1d_v, p1a_v, exc_v,
             sh_hout, sh_dentot, sem):
    cid = lax.axis_index("c")
    sid = lax.axis_index("s")

    pltpu.sync_copy(asrc_hbm, asrc_v)
    pltpu.sync_copy(adst_hbm, adst_v)
    pltpu.sync_copy(mh_hbm, mh_v)
    mh = mh_v[...]
    z16 = jnp.zeros((16,), jnp.float32)

    def zden(i, c):
        den_v[pl.ds(i * 16, 16)] = z16
        return c
    lax.fori_loop(0, N_PAD // 16, zden, 0)
    pltpu.sync_copy(den_v.at[pl.ds(0, _ROWS_PT)],
                    sh_dentot.at[pl.ds(sid * _ROWS_PT, _ROWS_PT)])

    def zrows(i, c):
        for k in range(HID // 16):
            rows_v[i, pl.ds(k * 16, 16)] = z16
        return c
    lax.fori_loop(0, _P2_CHUNK, zrows, 0)
    for b in range(_ROWS_PT // _P2_CHUNK):
        pltpu.sync_copy(
            rows_v, sh_hout.at[pl.ds(sid * _ROWS_PT + b * _P2_CHUNK,
                                     _P2_CHUNK)])
    plsc.subcore_barrier()

    def p1_chunk(c, carry):
        base = sid * _P1_EPT + c * _P1_CHUNK
        pltpu.sync_copy(src_hbm.at[pl.ds(base, _P1_CHUNK)], p1s_v)
        pltpu.sync_copy(dst_hbm.at[pl.ds(base, _P1_CHUNK)], p1d_v)
        pltpu.sync_copy(ae_hbm.at[pl.ds(base, _P1_CHUNK)], p1a_v)

        def inner(j, cc):
            s16 = p1s_v[pl.ds(j * 16, 16)]
            d16 = p1d_v[pl.ds(j * 16, 16)]
            ae16 = p1a_v[pl.ds(j * 16, 16)]
            al = (plsc.load_gather(asrc_v, [s16])
                  + plsc.load_gather(adst_v, [d16]) + ae16)
            al = jnp.maximum(al, 0.2 * al)
            exc_v[pl.ds(j * 16, 16)] = jnp.exp(al - mh)
            return cc
        lax.fori_loop(0, _P1_CHUNK // 16, inner, 0)
        pltpu.sync_copy(exc_v, sh_dentot.at[p1d_v], add=True)
        return carry
    lax.fori_loop(0, _P1_EPT // _P1_CHUNK, p1_chunk, 0)
    plsc.subcore_barrier()
    pltpu.sync_copy(sh_dentot, den_v)

    def p2_chunk(c, carry):
        base = (cid * 16 + sid) * _P2_EPT + c * _P2_CHUNK
        pltpu.sync_copy(src_hbm.at[pl.ds(base, _P2_CHUNK)], sidx_v)
        pltpu.sync_copy(dst_hbm.at[pl.ds(base, _P2_CHUNK)], didx_v)
        pltpu.sync_copy(ae_hbm.at[pl.ds(base, _P2_CHUNK)], aec_v)
        pltpu.async_copy(xl_hbm.at[sidx_v], rows_v, sem).wait()

        def attn(j, cc):
            s16 = sidx_v[pl.ds(j * 16, 16)]
            d16 = didx_v[pl.ds(j * 16, 16)]
            ae16 = aec_v[pl.ds(j * 16, 16)]
            al = (plsc.load_gather(asrc_v, [s16])
                  + plsc.load_gather(adst_v, [d16]) + ae16)
            al = jnp.maximum(al, 0.2 * al)
            ex = jnp.exp(al - mh)
            dn = plsc.load_gather(den_v, [d16])
            a_v[pl.ds(j * 16, 16)] = ex / (dn + 1e-16)
            return cc
        lax.fori_loop(0, _P2_CHUNK // 16, attn, 0)

        def scale(e, cc):
            ab = plsc.load_gather(a_v, [jnp.full((16,), e, jnp.int32)])
            for k in range(HID // 16):
                rows_v[e, pl.ds(k * 16, 16)] = (
                    rows_v[e, pl.ds(k * 16, 16)] * ab)
            return cc
        lax.fori_loop(0, _P2_CHUNK, scale, 0)
        pltpu.sync_copy(rows_v, sh_hout.at[didx_v], add=True)
        return carry
    lax.fori_loop(0, _P2_EPT // _P2_CHUNK, p2_chunk, 0)
    plsc.subcore_barrier()
    pltpu.sync_copy(sh_hout.at[pl.ds(sid * _ROWS_PT, _ROWS_PT)],
                    out_hbm.at[cid, pl.ds(sid * _ROWS_PT, _ROWS_PT)])


def _message_passing(src, dst, ae, asrc, adst, mh16, xl):
    mesh = plsc.VectorSubcoreMesh(core_axis_name="c", subcore_axis_name="s")
    f32, i32 = jnp.float32, jnp.int32
    k = pl.kernel(
        _mp_body,
        out_type=jax.ShapeDtypeStruct((2, N_PAD, HID), f32),
        mesh=mesh,
        compiler_params=pltpu.CompilerParams(needs_layout_passes=False),
        scratch_types=[
            pltpu.VMEM((N_NODES,), f32),
            pltpu.VMEM((N_NODES,), f32),
            pltpu.VMEM((N_PAD,), f32),
            pltpu.VMEM((_P2_CHUNK, HID), f32),
            pltpu.VMEM((_P2_CHUNK,), i32),
            pltpu.VMEM((_P2_CHUNK,), i32),
            pltpu.VMEM((_P2_CHUNK,), f32),
            pltpu.VMEM((_P2_CHUNK,), f32),
            pltpu.VMEM((16,), f32),
            pltpu.VMEM((_P1_CHUNK,), i32),
            pltpu.VMEM((_P1_CHUNK,), i32),
            pltpu.VMEM((_P1_CHUNK,), f32),
            pltpu.VMEM((_P1_CHUNK,), f32),
            pltpu.VMEM_SHARED((N_PAD, HID), f32),
            pltpu.VMEM_SHARED((N_PAD,), f32),
            pltpu.SemaphoreType.DMA,
        ],
    )
    return k(src, dst, ae, asrc, adst, mh16, xl)


def _finish_body(p0_ref, p1_ref, bg_ref, n2g_ref, h0_ref, p_ref):
    hout = p0_ref[...] + p1_ref[...] + bg_ref[...]
    n = hout.shape[0]
    bs = p_ref.shape[0]
    gi = lax.broadcasted_iota(jnp.int32, (n, bs), 1)
    oh = (n2g_ref[...] == gi).astype(jnp.float32)
    dn = (((0,), (0,)), ((), ()))
    psum = lax.dot_general(oh, hout, dn, preferred_element_type=jnp.float32)
    cnt = lax.dot_general(oh, jnp.ones((n, 1), jnp.float32), dn,
                          preferred_element_type=jnp.float32)
    p_ref[...] = jax.nn.relu(psum / jnp.maximum(cnt, 1.0))
    ri = lax.broadcasted_iota(jnp.int32, (n, bs), 0)
    oh0 = (ri == gi * (n // bs)).astype(jnp.float32)
    h0_ref[...] = lax.dot_general(oh0, hout, dn,
                                  preferred_element_type=jnp.float32)


def _finish(part0, part1, bg, n2g):
    return pl.pallas_call(
        _finish_body,
        out_shape=[
            jax.ShapeDtypeStruct((BS, HID), jnp.float32),
            jax.ShapeDtypeStruct((BS, HID), jnp.float32),
        ],
    )(part0, part1, bg, n2g)


def kernel(qa_emb, x, node_ids, node_types, node_scores, edge_index,
           edge_type, edge_attr, node2graph, W_nt, b_nt, W_x2h, b_x2h, W_e1,
           b_e1, W_e2, b_e2, W_gat, att_src, att_dst, W_edge, att_edge,
           b_gat):
    xl, a_s, a_d, amax = _node_enc(
        x, qa_emb, node_types, node_scores, W_nt, b_nt.reshape(1, -1),
        W_x2h, b_x2h.reshape(1, -1), W_gat,
        att_src.reshape(-1, 1), att_dst.reshape(-1, 1))
    ae, aemax = _edge_enc(edge_attr, W_e1, b_e1.reshape(1, -1), W_e2,
                          b_e2.reshape(1, -1), W_edge,
                          att_edge.reshape(-1, 1))
    msum = amax[0, 0] + amax[0, 1] + aemax[0, 0]
    mh = jnp.maximum(msum, 0.2 * msum)
    mh16 = jnp.broadcast_to(mh, (16,))
    hpart = _message_passing(edge_index[0], edge_index[1], ae.reshape(-1),
                             a_s.reshape(-1), a_d.reshape(-1), mh16, xl)
    h0, p = _finish(hpart[0, :N_NODES], hpart[1, :N_NODES],
                    b_gat.reshape(1, -1), node2graph.reshape(-1, 1))
    return (h0, p)


# speedup vs baseline: 13.2675x; 1.7684x over previous
import jax
import jax.numpy as jnp
from jax import lax
from jax.experimental import pallas as pl
from jax.experimental.pallas import tpu as pltpu
from jax.experimental.pallas import tpu_sc as plsc

N_NODES = 10000
N_EDGES = 320000
HID = 128
BS = 10
N_PAD = 10240


def _node_enc_body(x_ref, qa_ref, nt_ref, ns_ref, wnt_ref, bnt_ref, wx_ref,
                   bx_ref, wg_ref, asv_ref, adv_ref,
                   xl_ref, asrc_ref, adst_ref, amax_ref):
    n = x_ref.shape[0]
    bs = qa_ref.shape[0]
    npb = n // bs
    ri = lax.broadcasted_iota(jnp.int32, (n, bs), 0)
    ci = lax.broadcasted_iota(jnp.int32, (n, bs), 1)
    ohq = (ri == ci * npb).astype(jnp.float32)
    xq = jnp.dot(ohq, qa_ref[...], preferred_element_type=jnp.float32)
    ri1 = lax.broadcasted_iota(jnp.int32, (n, 1), 0)
    x = jnp.where((ri1 % npb) == 0, xq, x_ref[...])
    extras = (jnp.dot(jnp.concatenate([nt_ref[...], ns_ref[...]], axis=1),
                      wnt_ref[...], preferred_element_type=jnp.float32)
              + bnt_ref[...])
    xx = jax.nn.relu(jnp.concatenate([x, extras], axis=1))
    h = jax.nn.relu(jnp.dot(xx, wx_ref[...],
                            preferred_element_type=jnp.float32) + bx_ref[...])
    xl = jnp.dot(h, wg_ref[...], preferred_element_type=jnp.float32)
    xl_ref[...] = xl
    a_s = jnp.dot(xl, asv_ref[...], preferred_element_type=jnp.float32)
    a_d = jnp.dot(xl, adv_ref[...], preferred_element_type=jnp.float32)
    asrc_ref[...] = a_s
    adst_ref[...] = a_d
    amax_ref[...] = jnp.concatenate(
        [jnp.max(a_s).reshape(1, 1), jnp.max(a_d).reshape(1, 1)], axis=1)


def _node_enc(x, qa, nt, ns, wnt, bnt, wx, bx, wg, asv, adv):
    n = x.shape[0]
    return pl.pallas_call(
        _node_enc_body,
        out_shape=[
            jax.ShapeDtypeStruct((n, HID), jnp.float32),
            jax.ShapeDtypeStruct((n, 1), jnp.float32),
            jax.ShapeDtypeStruct((n, 1), jnp.float32),
            jax.ShapeDtypeStruct((1, 2), jnp.float32),
        ],
    )(x, qa, nt, ns, wnt, bnt, wx, bx, wg, asv, adv)


_EDGE_BLK = 2000


def _edge_enc_body(ea_ref, w1_ref, b1_ref, w2_ref, b2_ref, we_ref, aev_ref,
                   ae_ref, aemax_ref):
    t = jax.nn.relu(jnp.dot(ea_ref[...], w1_ref[...],
                            preferred_element_type=jnp.float32) + b1_ref[...])
    u = jax.nn.relu(jnp.dot(t.astype(jnp.bfloat16), w2_ref[...],
                            preferred_element_type=jnp.float32) + b2_ref[...])
    ve = jnp.dot(we_ref[...], aev_ref[...],
                 preferred_element_type=jnp.float32)
    ae = jnp.dot(u, ve, preferred_element_type=jnp.float32)
    ae_ref[...] = ae
    m = jnp.max(ae).reshape(1, 1)
    i = pl.program_id(0)

    @pl.when(i == 0)
    def _():
        aemax_ref[...] = m

    @pl.when(i > 0)
    def _():
        aemax_ref[...] = jnp.maximum(aemax_ref[...], m)


def _edge_enc(ea, w1, b1, w2, b2, we, aev):
    e, ein = ea.shape
    grid = e // _EDGE_BLK
    ea = ea.astype(jnp.bfloat16)
    w1 = w1.astype(jnp.bfloat16)
    w2 = w2.astype(jnp.bfloat16)
    return pl.pallas_call(
        _edge_enc_body,
        grid=(grid,),
        in_specs=[
            pl.BlockSpec((_EDGE_BLK, ein), lambda i: (i, 0)),
            pl.BlockSpec((ein, HID), lambda i: (0, 0)),
            pl.BlockSpec((1, HID), lambda i: (0, 0)),
            pl.BlockSpec((HID, HID), lambda i: (0, 0)),
            pl.BlockSpec((1, HID), lambda i: (0, 0)),
            pl.BlockSpec((HID, HID), lambda i: (0, 0)),
            pl.BlockSpec((HID, 1), lambda i: (0, 0)),
        ],
        out_specs=[
            pl.BlockSpec((_EDGE_BLK, 1), lambda i: (i, 0)),
            pl.BlockSpec((1, 1), lambda i: (0, 0)),
        ],
        out_shape=[
            jax.ShapeDtypeStruct((e, 1), jnp.float32),
            jax.ShapeDtypeStruct((1, 1), jnp.float32),
        ],
    )(ea, w1, b1, w2, b2, we, aev)


_CH = 80
_EPT = N_EDGES // 32
_NCH = _EPT // _CH
_ROWS_PT = N_PAD // 16


def _mp_body(pk_hbm, asrc_hbm, adst_hbm, mh_hbm, xl_hbm,
             outh_hbm, outd_hbm,
             asrc_v, adst_v, mh_v, zb_v,
             pk0, pk1, rows0, rows1, exc0, exc1,
             sh_hout, sh_dentot,
             pks0, pks1, gs0, gs1, s1a, s1b, s2a, s2b):
    cid = lax.axis_index("c")
    sid = lax.axis_index("s")
    cbase = (cid * 16 + sid) * _NCH

    pltpu.sync_copy(asrc_hbm, asrc_v)
    pltpu.sync_copy(adst_hbm, adst_v)
    pltpu.sync_copy(mh_hbm, mh_v)
    mh = mh_v[...]
    z16 = jnp.zeros((16,), jnp.float32)

    def zzb(i, c):
        zb_v[pl.ds(i * 16, 16)] = z16
        return c
    lax.fori_loop(0, _ROWS_PT // 16, zzb, 0)

    def zrows(i, c):
        for k in range(HID // 16):
            rows0[i, pl.ds(k * 16, 16)] = z16
        return c
    lax.fori_loop(0, _CH, zrows, 0)
    for b in range(_ROWS_PT // _CH):
        pltpu.sync_copy(rows0,
                        sh_hout.at[pl.ds(sid * _ROWS_PT + b * _CH, _CH)])
    pltpu.sync_copy(zb_v, sh_dentot.at[pl.ds(sid * _ROWS_PT, _ROWS_PT)])
    plsc.subcore_barrier()

    bufs = ((pk0, rows0, exc0, pks0, gs0, s1a, s2a),
            (pk1, rows1, exc1, pks1, gs1, s1b, s2b))

    def issue_pk(c, pk, pks):
        pltpu.async_copy(pk_hbm.at[cbase + c], pk, pks)

    issue_pk(0, pk0, pks0)
    issue_pk(1, pk1, pks1)

    def process(c, b, refill):
        pk, rows, exc, pks, gs, s1, s2 = bufs[b]
        pltpu.make_async_copy(pk_hbm.at[0], pk, pks).wait()
        gat = pltpu.async_copy(xl_hbm.at[pk.at[0]], rows, gs)

        def att(j, cc):
            s16 = pk[0, pl.ds(j * 16, 16)]
            d16 = pk[1, pl.ds(j * 16, 16)]
            ae16 = plsc.bitcast(pk[2, pl.ds(j * 16, 16)], jnp.float32)
            al = (plsc.load_gather(asrc_v, [s16])
                  + plsc.load_gather(adst_v, [d16]) + ae16)
            al = jnp.maximum(al, 0.2 * al)
            exc[pl.ds(j * 16, 16)] = jnp.exp(al - mh)
            return cc
        lax.fori_loop(0, _CH // 16, att, 0)
        gat.wait()

        def scale8(i, cc):
            base = i * 8
            ebs = [plsc.load_gather(exc, [jnp.full((16,), base + q, jnp.int32)])
                   for q in range(8)]
            for q in range(8):
                for k in range(HID // 16):
                    rows[base + q, pl.ds(k * 16, 16)] = (
                        rows[base + q, pl.ds(k * 16, 16)] * ebs[q])
            return cc
        lax.fori_loop(0, _CH // 8, scale8, 0)
        d1 = pltpu.async_copy(rows, sh_hout.at[pk.at[1]], s1, add=True)
        d2 = pltpu.async_copy(exc, sh_dentot.at[pk.at[1]], s2, add=True)
        d1.wait()
        d2.wait()
        if refill:
            @pl.when(c + 2 < _NCH)
            def _():
                issue_pk(c + 2, pk, pks)

    def pair(cp, carry):
        process(2 * cp, 0, True)
        process(2 * cp + 1, 1, True)
        return carry
    lax.fori_loop(0, _NCH // 2, pair, 0)
    process(_NCH - 1, 0, False)

    plsc.subcore_barrier()
    pltpu.sync_copy(sh_hout.at[pl.ds(sid * _ROWS_PT, _ROWS_PT)],
                    outh_hbm.at[cid, pl.ds(sid * _ROWS_PT, _ROWS_PT)])
    pltpu.sync_copy(sh_dentot.at[pl.ds(sid * _ROWS_PT, _ROWS_PT)],
                    outd_hbm.at[cid, pl.ds(sid * _ROWS_PT, _ROWS_PT)])


def _message_passing(pk, asrc, adst, mh16, xl):
    mesh = plsc.VectorSubcoreMesh(core_axis_name="c", subcore_axis_name="s")
    f32, i32 = jnp.float32, jnp.int32
    k = pl.kernel(
        _mp_body,
        out_type=[
            jax.ShapeDtypeStruct((2, N_PAD, HID), f32),
            jax.ShapeDtypeStruct((2, N_PAD), f32),
        ],
        mesh=mesh,
        compiler_params=pltpu.CompilerParams(needs_layout_passes=False),
        scratch_types=[
            pltpu.VMEM((N_NODES,), f32),
            pltpu.VMEM((N_NODES,), f32),
            pltpu.VMEM((16,), f32),
            pltpu.VMEM((_ROWS_PT,), f32),
            pltpu.VMEM((3, _CH), i32),
            pltpu.VMEM((3, _CH), i32),
            pltpu.VMEM((_CH, HID), f32),
            pltpu.VMEM((_CH, HID), f32),
            pltpu.VMEM((_CH,), f32),
            pltpu.VMEM((_CH,), f32),
            pltpu.VMEM_SHARED((N_PAD, HID), f32),
            pltpu.VMEM_SHARED((N_PAD,), f32),
            pltpu.SemaphoreType.DMA,
            pltpu.SemaphoreType.DMA,
            pltpu.SemaphoreType.DMA,
            pltpu.SemaphoreType.DMA,
            pltpu.SemaphoreType.DMA,
            pltpu.SemaphoreType.DMA,
            pltpu.SemaphoreType.DMA,
            pltpu.SemaphoreType.DMA,
        ],
    )
    return k(pk, asrc, adst, mh16, xl)


def _finish_body(p0_ref, p1_ref, d0_ref, d1_ref, bg_ref, n2g_ref,
                 h0_ref, p_ref):
    den = d0_ref[...] + d1_ref[...] + 1e-16
    hout = (p0_ref[...] + p1_ref[...]) / den + bg_ref[...]
    n = hout.shape[0]
    bs = p_ref.shape[0]
    gi = lax.broadcasted_iota(jnp.int32, (n, bs), 1)
    oh = (n2g_ref[...] == gi).astype(jnp.float32)
    dn = (((0,), (0,)), ((), ()))
    psum = lax.dot_general(oh, hout, dn, preferred_element_type=jnp.float32)
    cnt = lax.dot_general(oh, jnp.ones((n, 1), jnp.float32), dn,
                          preferred_element_type=jnp.float32)
    p_ref[...] = jax.nn.relu(psum / jnp.maximum(cnt, 1.0))
    ri = lax.broadcasted_iota(jnp.int32, (n, bs), 0)
    oh0 = (ri == gi * (n // bs)).astype(jnp.float32)
    h0_ref[...] = lax.dot_general(oh0, hout, dn,
                                  preferred_element_type=jnp.float32)


def _finish(part0, part1, d0, d1, bg, n2g):
    return pl.pallas_call(
        _finish_body,
        out_shape=[
            jax.ShapeDtypeStruct((BS, HID), jnp.float32),
            jax.ShapeDtypeStruct((BS, HID), jnp.float32),
        ],
    )(part0, part1, d0, d1, bg, n2g)


def kernel(qa_emb, x, node_ids, node_types, node_scores, edge_index,
           edge_type, edge_attr, node2graph, W_nt, b_nt, W_x2h, b_x2h, W_e1,
           b_e1, W_e2, b_e2, W_gat, att_src, att_dst, W_edge, att_edge,
           b_gat):
    xl, a_s, a_d, amax = _node_enc(
        x, qa_emb, node_types, node_scores, W_nt, b_nt.reshape(1, -1),
        W_x2h, b_x2h.reshape(1, -1), W_gat,
        att_src.reshape(-1, 1), att_dst.reshape(-1, 1))
    ae, aemax = _edge_enc(edge_attr, W_e1, b_e1.reshape(1, -1), W_e2,
                          b_e2.reshape(1, -1), W_edge,
                          att_edge.reshape(-1, 1))
    msum = amax[0, 0] + amax[0, 1] + aemax[0, 0]
    mh = jnp.maximum(msum, 0.2 * msum)
    mh16 = jnp.broadcast_to(mh, (16,))
    pk = jnp.stack([edge_index[0], edge_index[1],
                    lax.bitcast_convert_type(ae.reshape(-1), jnp.int32)])
    pk = pk.reshape(3, N_EDGES // 80, 80).transpose(1, 0, 2)
    hpart, dpart = _message_passing(pk, a_s.reshape(-1), a_d.reshape(-1),
                                    mh16, xl)
    h0, p = _finish(hpart[0, :N_NODES], hpart[1, :N_NODES],
                    dpart[0, :N_NODES].reshape(-1, 1),
                    dpart[1, :N_NODES].reshape(-1, 1),
                    b_gat.reshape(1, -1), node2graph.reshape(-1, 1))
    return (h0, p)


# speedup vs baseline: 16.0921x; 1.2129x over previous
import jax
import jax.numpy as jnp
from jax import lax
from jax.experimental import pallas as pl
from jax.experimental.pallas import tpu as pltpu
from jax.experimental.pallas import tpu_sc as plsc

N_NODES = 10000
N_EDGES = 320000
HID = 128
BS = 10
N_PAD = 10240


def _node_enc_body(x_ref, qa_ref, nt_ref, ns_ref, wnt_ref, bnt_ref, wx_ref,
                   bx_ref, wg_ref, asv_ref, adv_ref,
                   xl_ref, asrc_ref, adst_ref, amax_ref):
    n = x_ref.shape[0]
    bs = qa_ref.shape[0]
    npb = n // bs
    ri = lax.broadcasted_iota(jnp.int32, (n, bs), 0)
    ci = lax.broadcasted_iota(jnp.int32, (n, bs), 1)
    ohq = (ri == ci * npb).astype(jnp.float32)
    xq = jnp.dot(ohq, qa_ref[...], preferred_element_type=jnp.float32)
    ri1 = lax.broadcasted_iota(jnp.int32, (n, 1), 0)
    x = jnp.where((ri1 % npb) == 0, xq, x_ref[...])
    extras = (jnp.dot(jnp.concatenate([nt_ref[...], ns_ref[...]], axis=1),
                      wnt_ref[...], preferred_element_type=jnp.float32)
              + bnt_ref[...])
    xx = jax.nn.relu(jnp.concatenate([x, extras], axis=1))
    h = jax.nn.relu(jnp.dot(xx, wx_ref[...],
                            preferred_element_type=jnp.float32) + bx_ref[...])
    xl = jnp.dot(h, wg_ref[...], preferred_element_type=jnp.float32)
    xl_ref[...] = xl
    a_s = jnp.dot(xl, asv_ref[...], preferred_element_type=jnp.float32)
    a_d = jnp.dot(xl, adv_ref[...], preferred_element_type=jnp.float32)
    asrc_ref[...] = a_s
    adst_ref[...] = a_d
    amax_ref[...] = jnp.concatenate(
        [jnp.max(a_s).reshape(1, 1), jnp.max(a_d).reshape(1, 1)], axis=1)


def _node_enc(x, qa, nt, ns, wnt, bnt, wx, bx, wg, asv, adv):
    n = x.shape[0]
    return pl.pallas_call(
        _node_enc_body,
        out_shape=[
            jax.ShapeDtypeStruct((n, HID), jnp.float32),
            jax.ShapeDtypeStruct((n, 1), jnp.float32),
            jax.ShapeDtypeStruct((n, 1), jnp.float32),
            jax.ShapeDtypeStruct((1, 2), jnp.float32),
        ],
    )(x, qa, nt, ns, wnt, bnt, wx, bx, wg, asv, adv)


_EDGE_BLK = 8000


def _edge_enc_body(ea_ref, w1_ref, b1_ref, w2_ref, b2_ref, we_ref, aev_ref,
                   ae_ref, aemax_ref):
    t = jax.nn.relu(jnp.dot(ea_ref[...].astype(jnp.bfloat16), w1_ref[...],
                            preferred_element_type=jnp.float32) + b1_ref[...])
    u = jax.nn.relu(jnp.dot(t.astype(jnp.bfloat16), w2_ref[...],
                            preferred_element_type=jnp.float32) + b2_ref[...])
    ve = jnp.dot(we_ref[...], aev_ref[...],
                 preferred_element_type=jnp.float32)
    ae = jnp.dot(u, ve, preferred_element_type=jnp.float32)
    ae_ref[...] = ae
    m = jnp.max(ae).reshape(1, 1)
    i = pl.program_id(0)

    @pl.when(i == 0)
    def _():
        aemax_ref[...] = m

    @pl.when(i > 0)
    def _():
        aemax_ref[...] = jnp.maximum(aemax_ref[...], m)


def _edge_enc(ea, w1, b1, w2, b2, we, aev):
    e, ein = ea.shape
    grid = e // _EDGE_BLK
    w1 = w1.astype(jnp.bfloat16)
    w2 = w2.astype(jnp.bfloat16)
    return pl.pallas_call(
        _edge_enc_body,
        grid=(grid,),
        in_specs=[
            pl.BlockSpec((_EDGE_BLK, ein), lambda i: (i, 0)),
            pl.BlockSpec((ein, HID), lambda i: (0, 0)),
            pl.BlockSpec((1, HID), lambda i: (0, 0)),
            pl.BlockSpec((HID, HID), lambda i: (0, 0)),
            pl.BlockSpec((1, HID), lambda i: (0, 0)),
            pl.BlockSpec((HID, HID), lambda i: (0, 0)),
            pl.BlockSpec((HID, 1), lambda i: (0, 0)),
        ],
        out_specs=[
            pl.BlockSpec((_EDGE_BLK, 1), lambda i: (i, 0)),
            pl.BlockSpec((1, 1), lambda i: (0, 0)),
        ],
        out_shape=[
            jax.ShapeDtypeStruct((e, 1), jnp.float32),
            jax.ShapeDtypeStruct((1, 1), jnp.float32),
        ],
    )(ea, w1, b1, w2, b2, we, aev)


_CH = 80
_EPT = N_EDGES // 32
_NCH = _EPT // _CH
_ROWS_PT = N_PAD // 16


def _mp_body(pk_hbm, ae_hbm, asrc_hbm, adst_hbm, mh_hbm, xl_hbm,
             outh_hbm, outd_hbm,
             asrc_v, adst_v, mh_v, zb_v,
             pk0, pk1, ae0, ae1, sd0, sd1, rows0, rows1, exc0, exc1,
             sh_hout, sh_dentot,
             pks0, pks1, aes0, aes1, gs0, gs1, s1a, s1b, s2a, s2b):
    cid = lax.axis_index("c")
    sid = lax.axis_index("s")
    cbase = (cid * 16 + sid) * _NCH

    pltpu.sync_copy(asrc_hbm, asrc_v)
    pltpu.sync_copy(adst_hbm, adst_v)
    pltpu.sync_copy(mh_hbm, mh_v)
    mh = mh_v[...]
    z16 = jnp.zeros((16,), jnp.float32)

    def zzb(i, c):
        zb_v[pl.ds(i * 16, 16)] = z16
        return c
    lax.fori_loop(0, _ROWS_PT // 16, zzb, 0)

    def zrows(i, c):
        for k in range(HID // 16):
            rows0[i, pl.ds(k * 16, 16)] = z16
        return c
    lax.fori_loop(0, _CH, zrows, 0)
    for b in range(_ROWS_PT // _CH):
        pltpu.sync_copy(rows0,
                        sh_hout.at[pl.ds(sid * _ROWS_PT + b * _CH, _CH)])
    pltpu.sync_copy(zb_v, sh_dentot.at[pl.ds(sid * _ROWS_PT, _ROWS_PT)])
    plsc.subcore_barrier()

    bufs = ((pk0, ae0, sd0, rows0, exc0, pks0, aes0, gs0, s1a, s2a),
            (pk1, ae1, sd1, rows1, exc1, pks1, aes1, gs1, s1b, s2b))

    def issue_pk(c, pk, ae, pks, aes):
        pltpu.async_copy(pk_hbm.at[cbase + c], pk, pks)
        pltpu.async_copy(ae_hbm.at[cbase + c], ae, aes)

    issue_pk(0, pk0, ae0, pks0, aes0)
    issue_pk(1, pk1, ae1, pks1, aes1)

    def process(c, b, refill, first):
        pk, ae, sd, rows, exc, pks, aes, gs, s1, s2 = bufs[b]
        pltpu.make_async_copy(pk_hbm.at[0], pk, pks).wait()
        if not first:
            pltpu.make_async_copy(rows, sh_hout.at[sd.at[1]], s1).wait()
            pltpu.make_async_copy(exc, sh_dentot.at[sd.at[1]], s2).wait()

        def unpack(j, cc):
            v = pk[pl.ds(j * 16, 16)]
            sd[0, pl.ds(j * 16, 16)] = jnp.bitwise_and(v, 16383)
            sd[1, pl.ds(j * 16, 16)] = lax.shift_right_logical(v, 14)
            return cc
        lax.fori_loop(0, _CH // 16, unpack, 0)
        gat = pltpu.async_copy(xl_hbm.at[sd.at[0]], rows, gs)
        pltpu.make_async_copy(ae_hbm.at[0], ae, aes).wait()

        def att(j, cc):
            s16 = sd[0, pl.ds(j * 16, 16)]
            d16 = sd[1, pl.ds(j * 16, 16)]
            ae16 = ae[pl.ds(j * 16, 16)]
            al = (plsc.load_gather(asrc_v, [s16])
                  + plsc.load_gather(adst_v, [d16]) + ae16)
            al = jnp.maximum(al, 0.2 * al)
            exc[pl.ds(j * 16, 16)] = jnp.exp(al - mh)
            return cc
        lax.fori_loop(0, _CH // 16, att, 0)
        gat.wait()

        def scale8(i, cc):
            base = i * 8
            ebs = [plsc.load_gather(exc, [jnp.full((16,), base + q, jnp.int32)])
                   for q in range(8)]
            for q in range(8):
                for k in range(HID // 16):
                    rows[base + q, pl.ds(k * 16, 16)] = (
                        rows[base + q, pl.ds(k * 16, 16)] * ebs[q])
            return cc
        lax.fori_loop(0, _CH // 8, scale8, 0)
        pltpu.async_copy(rows, sh_hout.at[sd.at[1]], s1, add=True)
        pltpu.async_copy(exc, sh_dentot.at[sd.at[1]], s2, add=True)
        if refill:
            @pl.when(c + 2 < _NCH)
            def _():
                issue_pk(c + 2, pk, ae, pks, aes)

    process(0, 0, True, True)
    process(1, 1, True, True)

    def pair(cp, carry):
        process(2 * cp, 0, True, False)
        process(2 * cp + 1, 1, True, False)
        return carry
    lax.fori_loop(1, _NCH // 2, pair, 0)
    process(_NCH - 1, 0, False, False)

    pltpu.make_async_copy(rows0, sh_hout.at[sd0.at[1]], s1a).wait()
    pltpu.make_async_copy(exc0, sh_dentot.at[sd0.at[1]], s2a).wait()
    pltpu.make_async_copy(rows1, sh_hout.at[sd1.at[1]], s1b).wait()
    pltpu.make_async_copy(exc1, sh_dentot.at[sd1.at[1]], s2b).wait()

    plsc.subcore_barrier()
    pltpu.sync_copy(sh_hout.at[pl.ds(sid * _ROWS_PT, _ROWS_PT)],
                    outh_hbm.at[cid, pl.ds(sid * _ROWS_PT, _ROWS_PT)])
    pltpu.sync_copy(sh_dentot.at[pl.ds(sid * _ROWS_PT, _ROWS_PT)],
                    outd_hbm.at[cid, pl.ds(sid * _ROWS_PT, _ROWS_PT)])


def _message_passing(pk, ae2d, asrc, adst, mh16, xl):
    mesh = plsc.VectorSubcoreMesh(core_axis_name="c", subcore_axis_name="s")
    f32, i32 = jnp.float32, jnp.int32
    k = pl.kernel(
        _mp_body,
        out_type=[
            jax.ShapeDtypeStruct((2, N_PAD, HID), f32),
            jax.ShapeDtypeStruct((2, N_PAD), f32),
        ],
        mesh=mesh,
        compiler_params=pltpu.CompilerParams(needs_layout_passes=False),
        scratch_types=[
            pltpu.VMEM((N_NODES,), f32),
            pltpu.VMEM((N_NODES,), f32),
            pltpu.VMEM((16,), f32),
            pltpu.VMEM((_ROWS_PT,), f32),
            pltpu.VMEM((_CH,), i32),
            pltpu.VMEM((_CH,), i32),
            pltpu.VMEM((_CH,), f32),
            pltpu.VMEM((_CH,), f32),
            pltpu.VMEM((2, _CH), i32),
            pltpu.VMEM((2, _CH), i32),
            pltpu.VMEM((_CH, HID), f32),
            pltpu.VMEM((_CH, HID), f32),
            pltpu.VMEM((_CH,), f32),
            pltpu.VMEM((_CH,), f32),
            pltpu.VMEM_SHARED((N_PAD, HID), f32),
            pltpu.VMEM_SHARED((N_PAD,), f32),
            pltpu.SemaphoreType.DMA,
            pltpu.SemaphoreType.DMA,
            pltpu.SemaphoreType.DMA,
            pltpu.SemaphoreType.DMA,
            pltpu.SemaphoreType.DMA,
            pltpu.SemaphoreType.DMA,
            pltpu.SemaphoreType.DMA,
            pltpu.SemaphoreType.DMA,
            pltpu.SemaphoreType.DMA,
            pltpu.SemaphoreType.DMA,
        ],
    )
    return k(pk, ae2d, asrc, adst, mh16, xl)


def _finish_body(p0_ref, p1_ref, d0_ref, d1_ref, bg_ref, n2g_ref,
                 h0_ref, p_ref):
    den = d0_ref[...] + d1_ref[...] + 1e-16
    hout = (p0_ref[...] + p1_ref[...]) / den + bg_ref[...]
    n = hout.shape[0]
    bs = p_ref.shape[0]
    gi = lax.broadcasted_iota(jnp.int32, (n, bs), 1)
    oh = (n2g_ref[...] == gi).astype(jnp.float32)
    dn = (((0,), (0,)), ((), ()))
    psum = lax.dot_general(oh, hout, dn, preferred_element_type=jnp.float32)
    cnt = lax.dot_general(oh, jnp.ones((n, 1), jnp.float32), dn,
                          preferred_element_type=jnp.float32)
    p_ref[...] = jax.nn.relu(psum / jnp.maximum(cnt, 1.0))
    ri = lax.broadcasted_iota(jnp.int32, (n, bs), 0)
    oh0 = (ri == gi * (n // bs)).astype(jnp.float32)
    h0_ref[...] = lax.dot_general(oh0, hout, dn,
                                  preferred_element_type=jnp.float32)


def _finish(part0, part1, d0, d1, bg, n2g):
    return pl.pallas_call(
        _finish_body,
        out_shape=[
            jax.ShapeDtypeStruct((BS, HID), jnp.float32),
            jax.ShapeDtypeStruct((BS, HID), jnp.float32),
        ],
    )(part0, part1, d0, d1, bg, n2g)


def kernel(qa_emb, x, node_ids, node_types, node_scores, edge_index,
           edge_type, edge_attr, node2graph, W_nt, b_nt, W_x2h, b_x2h, W_e1,
           b_e1, W_e2, b_e2, W_gat, att_src, att_dst, W_edge, att_edge,
           b_gat):
    xl, a_s, a_d, amax = _node_enc(
        x, qa_emb, node_types, node_scores, W_nt, b_nt.reshape(1, -1),
        W_x2h, b_x2h.reshape(1, -1), W_gat,
        att_src.reshape(-1, 1), att_dst.reshape(-1, 1))
    ae, aemax = _edge_enc(edge_attr, W_e1, b_e1.reshape(1, -1), W_e2,
                          b_e2.reshape(1, -1), W_edge,
                          att_edge.reshape(-1, 1))
    msum = amax[0, 0] + amax[0, 1] + aemax[0, 0]
    mh = jnp.maximum(msum, 0.2 * msum)
    mh16 = jnp.broadcast_to(mh, (16,))
    src = edge_index[0].astype(jnp.int32)
    dst = edge_index[1].astype(jnp.int32)
    pk = (src | (dst << 14)).reshape(N_EDGES // _CH, _CH)
    ae2d = ae.reshape(N_EDGES // _CH, _CH)
    hpart, dpart = _message_passing(pk, ae2d, a_s.reshape(-1),
                                    a_d.reshape(-1), mh16, xl)
    h0, p = _finish(hpart[0, :N_NODES], hpart[1, :N_NODES],
                    dpart[0, :N_NODES].reshape(-1, 1),
                    dpart[1, :N_NODES].reshape(-1, 1),
                    b_gat.reshape(1, -1), node2graph.reshape(-1, 1))
    return (h0, p)


# speedup vs baseline: 16.4034x; 1.0193x over previous
import jax
import jax.numpy as jnp
from jax import lax
from jax.experimental import pallas as pl
from jax.experimental.pallas import tpu as pltpu
from jax.experimental.pallas import tpu_sc as plsc

N_NODES = 10000
N_EDGES = 320000
HID = 128
BS = 10
N_PAD = 10240


def _node_enc_body(x_ref, qa_ref, nt_ref, ns_ref, wnt_ref, bnt_ref, wx_ref,
                   bx_ref, wg_ref, asv_ref, adv_ref,
                   xl_ref, asrc_ref, adst_ref, amax_ref):
    n = x_ref.shape[0]
    bs = qa_ref.shape[0]
    npb = n // bs
    ri = lax.broadcasted_iota(jnp.int32, (n, bs), 0)
    ci = lax.broadcasted_iota(jnp.int32, (n, bs), 1)
    ohq = (ri == ci * npb).astype(jnp.float32)
    xq = jnp.dot(ohq, qa_ref[...], preferred_element_type=jnp.float32)
    ri1 = lax.broadcasted_iota(jnp.int32, (n, 1), 0)
    x = jnp.where((ri1 % npb) == 0, xq, x_ref[...])
    extras = (jnp.dot(jnp.concatenate([nt_ref[...], ns_ref[...]], axis=1),
                      wnt_ref[...], preferred_element_type=jnp.float32)
              + bnt_ref[...])
    xx = jax.nn.relu(jnp.concatenate([x, extras], axis=1))
    h = jax.nn.relu(jnp.dot(xx, wx_ref[...],
                            preferred_element_type=jnp.float32) + bx_ref[...])
    xl = jnp.dot(h, wg_ref[...], preferred_element_type=jnp.float32)
    xl_ref[...] = xl
    a_s = jnp.dot(xl, asv_ref[...], preferred_element_type=jnp.float32)
    a_d = jnp.dot(xl, adv_ref[...], preferred_element_type=jnp.float32)
    asrc_ref[...] = a_s
    adst_ref[...] = a_d
    amax_ref[...] = jnp.concatenate(
        [jnp.max(a_s).reshape(1, 1), jnp.max(a_d).reshape(1, 1)], axis=1)


def _node_enc(x, qa, nt, ns, wnt, bnt, wx, bx, wg, asv, adv):
    n = x.shape[0]
    return pl.pallas_call(
        _node_enc_body,
        out_shape=[
            jax.ShapeDtypeStruct((n, HID), jnp.float32),
            jax.ShapeDtypeStruct((n, 1), jnp.float32),
            jax.ShapeDtypeStruct((n, 1), jnp.float32),
            jax.ShapeDtypeStruct((1, 2), jnp.float32),
        ],
    )(x, qa, nt, ns, wnt, bnt, wx, bx, wg, asv, adv)


_EDGE_BLK = 8000


def _edge_enc_body(ea_ref, w1_ref, b1_ref, w2_ref, b2_ref, we_ref, aev_ref,
                   ae_ref, aemax_ref):
    t = jax.nn.relu(jnp.dot(ea_ref[...], w1_ref[...],
                            preferred_element_type=jnp.float32) + b1_ref[...])
    u = jax.nn.relu(jnp.dot(t.astype(jnp.bfloat16), w2_ref[...],
                            preferred_element_type=jnp.float32) + b2_ref[...])
    ve = jnp.dot(we_ref[...], aev_ref[...],
                 preferred_element_type=jnp.float32)
    ae = jnp.dot(u, ve, preferred_element_type=jnp.float32)
    ae_ref[...] = ae
    m = jnp.max(ae).reshape(1, 1)
    i = pl.program_id(0)

    @pl.when(i == 0)
    def _():
        aemax_ref[...] = m

    @pl.when(i > 0)
    def _():
        aemax_ref[...] = jnp.maximum(aemax_ref[...], m)


def _edge_enc(ea, w1, b1, w2, b2, we, aev):
    e, ein = ea.shape
    grid = e // _EDGE_BLK
    ea = ea.astype(jnp.bfloat16)
    w1 = w1.astype(jnp.bfloat16)
    w2 = w2.astype(jnp.bfloat16)
    return pl.pallas_call(
        _edge_enc_body,
        grid=(grid,),
        in_specs=[
            pl.BlockSpec((_EDGE_BLK, ein), lambda i: (i, 0)),
            pl.BlockSpec((ein, HID), lambda i: (0, 0)),
            pl.BlockSpec((1, HID), lambda i: (0, 0)),
            pl.BlockSpec((HID, HID), lambda i: (0, 0)),
            pl.BlockSpec((1, HID), lambda i: (0, 0)),
            pl.BlockSpec((HID, HID), lambda i: (0, 0)),
            pl.BlockSpec((HID, 1), lambda i: (0, 0)),
        ],
        out_specs=[
            pl.BlockSpec((_EDGE_BLK, 1), lambda i: (i, 0)),
            pl.BlockSpec((1, 1), lambda i: (0, 0)),
        ],
        out_shape=[
            jax.ShapeDtypeStruct((e, 1), jnp.float32),
            jax.ShapeDtypeStruct((1, 1), jnp.float32),
        ],
    )(ea, w1, b1, w2, b2, we, aev)


_CH = 80
_EPT = N_EDGES // 32
_NCH = _EPT // _CH
_ROWS_PT = N_PAD // 16


def _mp_body(pk_hbm, ae_hbm, asrc_hbm, adst_hbm, mh_hbm, xl_hbm,
             outh_hbm, outd_hbm,
             asrc_v, adst_v, mh_v, zb_v,
             pk0, pk1, ae0, ae1, sd0, sd1, rows0, rows1, exc0, exc1,
             sh_hout, sh_dentot,
             pks0, pks1, aes0, aes1, gs0, gs1, s1a, s1b, s2a, s2b):
    cid = lax.axis_index("c")
    sid = lax.axis_index("s")
    cbase = (cid * 16 + sid) * _NCH

    pltpu.sync_copy(asrc_hbm, asrc_v)
    pltpu.sync_copy(adst_hbm, adst_v)
    pltpu.sync_copy(mh_hbm, mh_v)
    mh = mh_v[...]
    z16 = jnp.zeros((16,), jnp.float32)

    def zzb(i, c):
        zb_v[pl.ds(i * 16, 16)] = z16
        return c
    lax.fori_loop(0, _ROWS_PT // 16, zzb, 0)

    def zrows(i, c):
        for k in range(HID // 16):
            rows0[i, pl.ds(k * 16, 16)] = z16
        return c
    lax.fori_loop(0, _CH, zrows, 0)
    for b in range(_ROWS_PT // _CH):
        pltpu.sync_copy(rows0,
                        sh_hout.at[pl.ds(sid * _ROWS_PT + b * _CH, _CH)])
    pltpu.sync_copy(zb_v, sh_dentot.at[pl.ds(sid * _ROWS_PT, _ROWS_PT)])
    plsc.subcore_barrier()

    bufs = ((pk0, ae0, sd0, rows0, exc0, pks0, aes0, gs0, s1a, s2a),
            (pk1, ae1, sd1, rows1, exc1, pks1, aes1, gs1, s1b, s2b))

    def issue_pk(c, pk, ae, pks, aes):
        pltpu.async_copy(pk_hbm.at[cbase + c], pk, pks)
        pltpu.async_copy(ae_hbm.at[cbase + c], ae, aes)

    issue_pk(0, pk0, ae0, pks0, aes0)
    issue_pk(1, pk1, ae1, pks1, aes1)

    def process(c, b, refill, first):
        pk, ae, sd, rows, exc, pks, aes, gs, s1, s2 = bufs[b]
        pltpu.make_async_copy(pk_hbm.at[0], pk, pks).wait()
        if not first:
            pltpu.make_async_copy(rows, sh_hout.at[sd.at[1]], s1).wait()
            pltpu.make_async_copy(exc, sh_dentot.at[sd.at[1]], s2).wait()

        def unpack(j, cc):
            v = pk[pl.ds(j * 16, 16)]
            sd[0, pl.ds(j * 16, 16)] = jnp.bitwise_and(v, 16383)
            sd[1, pl.ds(j * 16, 16)] = lax.shift_right_logical(v, 14)
            return cc
        lax.fori_loop(0, _CH // 16, unpack, 0)
        gat = pltpu.async_copy(xl_hbm.at[sd.at[0]], rows, gs)
        pltpu.make_async_copy(ae_hbm.at[0], ae, aes).wait()

        def att(j, cc):
            s16 = sd[0, pl.ds(j * 16, 16)]
            d16 = sd[1, pl.ds(j * 16, 16)]
            ae16 = ae[pl.ds(j * 16, 16)]
            al = (plsc.load_gather(asrc_v, [s16])
                  + plsc.load_gather(adst_v, [d16]) + ae16)
            al = jnp.maximum(al, 0.2 * al)
            exc[pl.ds(j * 16, 16)] = jnp.exp(al - mh)
            return cc
        lax.fori_loop(0, _CH // 16, att, 0)
        gat.wait()

        def scale8(i, cc):
            base = i * 8
            ebs = [plsc.load_gather(exc, [jnp.full((16,), base + q, jnp.int32)])
                   for q in range(8)]
            for q in range(8):
                for k in range(HID // 16):
                    rows[base + q, pl.ds(k * 16, 16)] = (
                        rows[base + q, pl.ds(k * 16, 16)] * ebs[q])
            return cc
        lax.fori_loop(0, _CH // 8, scale8, 0)
        pltpu.async_copy(rows, sh_hout.at[sd.at[1]], s1, add=True)
        pltpu.async_copy(exc, sh_dentot.at[sd.at[1]], s2, add=True)
        if refill:
            @pl.when(c + 2 < _NCH)
            def _():
                issue_pk(c + 2, pk, ae, pks, aes)

    process(0, 0, True, True)
    process(1, 1, True, True)

    def pair(cp, carry):
        process(2 * cp, 0, True, False)
        process(2 * cp + 1, 1, True, False)
        return carry
    lax.fori_loop(1, _NCH // 2, pair, 0)
    process(_NCH - 1, 0, False, False)

    pltpu.make_async_copy(rows0, sh_hout.at[sd0.at[1]], s1a).wait()
    pltpu.make_async_copy(exc0, sh_dentot.at[sd0.at[1]], s2a).wait()
    pltpu.make_async_copy(rows1, sh_hout.at[sd1.at[1]], s1b).wait()
    pltpu.make_async_copy(exc1, sh_dentot.at[sd1.at[1]], s2b).wait()

    plsc.subcore_barrier()
    pltpu.sync_copy(sh_hout.at[pl.ds(sid * _ROWS_PT, _ROWS_PT)],
                    outh_hbm.at[cid, pl.ds(sid * _ROWS_PT, _ROWS_PT)])
    pltpu.sync_copy(sh_dentot.at[pl.ds(sid * _ROWS_PT, _ROWS_PT)],
                    outd_hbm.at[cid, pl.ds(sid * _ROWS_PT, _ROWS_PT)])


def _message_passing(pk, ae2d, asrc, adst, mh16, xl):
    mesh = plsc.VectorSubcoreMesh(core_axis_name="c", subcore_axis_name="s")
    f32, i32 = jnp.float32, jnp.int32
    k = pl.kernel(
        _mp_body,
        out_type=[
            jax.ShapeDtypeStruct((2, N_PAD, HID), f32),
            jax.ShapeDtypeStruct((2, N_PAD), f32),
        ],
        mesh=mesh,
        compiler_params=pltpu.CompilerParams(needs_layout_passes=False),
        scratch_types=[
            pltpu.VMEM((N_NODES,), f32),
            pltpu.VMEM((N_NODES,), f32),
            pltpu.VMEM((16,), f32),
            pltpu.VMEM((_ROWS_PT,), f32),
            pltpu.VMEM((_CH,), i32),
            pltpu.VMEM((_CH,), i32),
            pltpu.VMEM((_CH,), f32),
            pltpu.VMEM((_CH,), f32),
            pltpu.VMEM((2, _CH), i32),
            pltpu.VMEM((2, _CH), i32),
            pltpu.VMEM((_CH, HID), f32),
            pltpu.VMEM((_CH, HID), f32),
            pltpu.VMEM((_CH,), f32),
            pltpu.VMEM((_CH,), f32),
            pltpu.VMEM_SHARED((N_PAD, HID), f32),
            pltpu.VMEM_SHARED((N_PAD,), f32),
            pltpu.SemaphoreType.DMA,
            pltpu.SemaphoreType.DMA,
            pltpu.SemaphoreType.DMA,
            pltpu.SemaphoreType.DMA,
            pltpu.SemaphoreType.DMA,
            pltpu.SemaphoreType.DMA,
            pltpu.SemaphoreType.DMA,
            pltpu.SemaphoreType.DMA,
            pltpu.SemaphoreType.DMA,
            pltpu.SemaphoreType.DMA,
        ],
    )
    return k(pk, ae2d, asrc, adst, mh16, xl)


def _finish_body(p0_ref, p1_ref, d0_ref, d1_ref, bg_ref, n2g_ref,
                 h0_ref, p_ref):
    den = d0_ref[...] + d1_ref[...] + 1e-16
    hout = (p0_ref[...] + p1_ref[...]) / den + bg_ref[...]
    n = hout.shape[0]
    bs = p_ref.shape[0]
    gi = lax.broadcasted_iota(jnp.int32, (n, bs), 1)
    oh = (n2g_ref[...] == gi).astype(jnp.float32)
    dn = (((0,), (0,)), ((), ()))
    psum = lax.dot_general(oh, hout, dn, preferred_element_type=jnp.float32)
    cnt = lax.dot_general(oh, jnp.ones((n, 1), jnp.float32), dn,
                          preferred_element_type=jnp.float32)
    p_ref[...] = jax.nn.relu(psum / jnp.maximum(cnt, 1.0))
    ri = lax.broadcasted_iota(jnp.int32, (n, bs), 0)
    oh0 = (ri == gi * (n // bs)).astype(jnp.float32)
    h0_ref[...] = lax.dot_general(oh0, hout, dn,
                                  preferred_element_type=jnp.float32)


def _finish(part0, part1, d0, d1, bg, n2g):
    return pl.pallas_call(
        _finish_body,
        out_shape=[
            jax.ShapeDtypeStruct((BS, HID), jnp.float32),
            jax.ShapeDtypeStruct((BS, HID), jnp.float32),
        ],
    )(part0, part1, d0, d1, bg, n2g)


def kernel(qa_emb, x, node_ids, node_types, node_scores, edge_index,
           edge_type, edge_attr, node2graph, W_nt, b_nt, W_x2h, b_x2h, W_e1,
           b_e1, W_e2, b_e2, W_gat, att_src, att_dst, W_edge, att_edge,
           b_gat):
    xl, a_s, a_d, amax = _node_enc(
        x, qa_emb, node_types, node_scores, W_nt, b_nt.reshape(1, -1),
        W_x2h, b_x2h.reshape(1, -1), W_gat,
        att_src.reshape(-1, 1), att_dst.reshape(-1, 1))
    ae, aemax = _edge_enc(edge_attr, W_e1, b_e1.reshape(1, -1), W_e2,
                          b_e2.reshape(1, -1), W_edge,
                          att_edge.reshape(-1, 1))
    msum = amax[0, 0] + amax[0, 1] + aemax[0, 0]
    mh = jnp.maximum(msum, 0.2 * msum)
    mh16 = jnp.broadcast_to(mh, (16,))
    src = edge_index[0].astype(jnp.int32)
    dst = edge_index[1].astype(jnp.int32)
    pk = (src | (dst << 14)).reshape(N_EDGES // _CH, _CH)
    ae2d = ae.reshape(N_EDGES // _CH, _CH)
    hpart, dpart = _message_passing(pk, ae2d, a_s.reshape(-1),
                                    a_d.reshape(-1), mh16, xl)
    h0, p = _finish(hpart[0, :N_NODES], hpart[1, :N_NODES],
                    dpart[0, :N_NODES].reshape(-1, 1),
                    dpart[1, :N_NODES].reshape(-1, 1),
                    b_gat.reshape(1, -1), node2graph.reshape(-1, 1))
    return (h0, p)


# speedup vs baseline: 16.6346x; 1.0141x over previous
import jax
import jax.numpy as jnp
from jax import lax
from jax.experimental import pallas as pl
from jax.experimental.pallas import tpu as pltpu
from jax.experimental.pallas import tpu_sc as plsc

N_NODES = 10000
N_EDGES = 320000
HID = 128
BS = 10
N_PAD = 10240


def _node_enc_body(x_ref, qa_ref, nt_ref, ns_ref, wnt_ref, bnt_ref, wx_ref,
                   bx_ref, wg_ref, asv_ref, adv_ref,
                   xl_ref, asrc_ref, adst_ref, amax_ref):
    n = x_ref.shape[0]
    bs = qa_ref.shape[0]
    npb = n // bs
    ri = lax.broadcasted_iota(jnp.int32, (n, bs), 0)
    ci = lax.broadcasted_iota(jnp.int32, (n, bs), 1)
    ohq = (ri == ci * npb).astype(jnp.float32)
    xq = jnp.dot(ohq, qa_ref[...], preferred_element_type=jnp.float32)
    ri1 = lax.broadcasted_iota(jnp.int32, (n, 1), 0)
    x = jnp.where((ri1 % npb) == 0, xq, x_ref[...])
    extras = (jnp.dot(jnp.concatenate([nt_ref[...], ns_ref[...]], axis=1),
                      wnt_ref[...], preferred_element_type=jnp.float32)
              + bnt_ref[...])
    xx = jax.nn.relu(jnp.concatenate([x, extras], axis=1))
    h = jax.nn.relu(jnp.dot(xx, wx_ref[...],
                            preferred_element_type=jnp.float32) + bx_ref[...])
    xl = jnp.dot(h, wg_ref[...], preferred_element_type=jnp.float32)
    xl_ref[...] = xl
    a_s = jnp.dot(xl, asv_ref[...], preferred_element_type=jnp.float32)
    a_d = jnp.dot(xl, adv_ref[...], preferred_element_type=jnp.float32)
    asrc_ref[...] = a_s
    adst_ref[...] = a_d
    amax_ref[...] = jnp.concatenate(
        [jnp.max(a_s).reshape(1, 1), jnp.max(a_d).reshape(1, 1)], axis=1)


def _node_enc(x, qa, nt, ns, wnt, bnt, wx, bx, wg, asv, adv):
    n = x.shape[0]
    return pl.pallas_call(
        _node_enc_body,
        out_shape=[
            jax.ShapeDtypeStruct((n, HID), jnp.float32),
            jax.ShapeDtypeStruct((n, 1), jnp.float32),
            jax.ShapeDtypeStruct((n, 1), jnp.float32),
            jax.ShapeDtypeStruct((1, 2), jnp.float32),
        ],
    )(x, qa, nt, ns, wnt, bnt, wx, bx, wg, asv, adv)


_EDGE_BLK = 16000


def _edge_enc_body(ea_ref, w1_ref, b1_ref, w2_ref, b2_ref, we_ref, aev_ref,
                   ae_ref, aemax_ref):
    t = jax.nn.relu(jnp.dot(ea_ref[...], w1_ref[...],
                            preferred_element_type=jnp.float32) + b1_ref[...])
    u = jax.nn.relu(jnp.dot(t.astype(jnp.bfloat16), w2_ref[...],
                            preferred_element_type=jnp.float32) + b2_ref[...])
    ve = jnp.dot(we_ref[...], aev_ref[...],
                 preferred_element_type=jnp.float32)
    ae = jnp.dot(u, ve, preferred_element_type=jnp.float32)
    ae_ref[...] = ae
    m = jnp.max(ae).reshape(1, 1)
    i = pl.program_id(0)

    @pl.when(i == 0)
    def _():
        aemax_ref[...] = m

    @pl.when(i > 0)
    def _():
        aemax_ref[...] = jnp.maximum(aemax_ref[...], m)


def _edge_enc(ea, w1, b1, w2, b2, we, aev):
    e, ein = ea.shape
    grid = e // _EDGE_BLK
    ea = ea.astype(jnp.bfloat16)
    w1 = w1.astype(jnp.bfloat16)
    w2 = w2.astype(jnp.bfloat16)
    return pl.pallas_call(
        _edge_enc_body,
        grid=(grid,),
        in_specs=[
            pl.BlockSpec((_EDGE_BLK, ein), lambda i: (i, 0)),
            pl.BlockSpec((ein, HID), lambda i: (0, 0)),
            pl.BlockSpec((1, HID), lambda i: (0, 0)),
            pl.BlockSpec((HID, HID), lambda i: (0, 0)),
            pl.BlockSpec((1, HID), lambda i: (0, 0)),
            pl.BlockSpec((HID, HID), lambda i: (0, 0)),
            pl.BlockSpec((HID, 1), lambda i: (0, 0)),
        ],
        out_specs=[
            pl.BlockSpec((_EDGE_BLK, 1), lambda i: (i, 0)),
            pl.BlockSpec((1, 1), lambda i: (0, 0)),
        ],
        out_shape=[
            jax.ShapeDtypeStruct((e, 1), jnp.float32),
            jax.ShapeDtypeStruct((1, 1), jnp.float32),
        ],
    )(ea, w1, b1, w2, b2, we, aev)


_CH = 80
_EPT = N_EDGES // 32
_NCH = _EPT // _CH
_ROWS_PT = N_PAD // 16


def _mp_body(pk_hbm, ae_hbm, asrc_hbm, adst_hbm, mh_hbm, xl_hbm,
             outh_hbm, outd_hbm,
             asrc_v, adst_v, mh_v, zb_v,
             pk0, pk1, ae0, ae1, sd0, sd1, rows0, rows1, exc0, exc1,
             sh_hout, sh_dentot,
             pks0, pks1, aes0, aes1, gs0, gs1, s1a, s1b, s2a, s2b):
    cid = lax.axis_index("c")
    sid = lax.axis_index("s")
    cbase = (cid * 16 + sid) * _NCH

    pltpu.sync_copy(asrc_hbm, asrc_v)
    pltpu.sync_copy(adst_hbm, adst_v)
    pltpu.sync_copy(mh_hbm, mh_v)
    mh = mh_v[...]
    z16 = jnp.zeros((16,), jnp.float32)

    def zzb(i, c):
        zb_v[pl.ds(i * 16, 16)] = z16
        return c
    lax.fori_loop(0, _ROWS_PT // 16, zzb, 0)

    def zrows(i, c):
        for k in range(HID // 16):
            rows0[i, pl.ds(k * 16, 16)] = z16
        return c
    lax.fori_loop(0, _CH, zrows, 0)
    for b in range(_ROWS_PT // _CH):
        pltpu.sync_copy(rows0,
                        sh_hout.at[pl.ds(sid * _ROWS_PT + b * _CH, _CH)])
    pltpu.sync_copy(zb_v, sh_dentot.at[pl.ds(sid * _ROWS_PT, _ROWS_PT)])
    plsc.subcore_barrier()

    bufs = ((pk0, ae0, sd0, rows0, exc0, pks0, aes0, gs0, s1a, s2a),
            (pk1, ae1, sd1, rows1, exc1, pks1, aes1, gs1, s1b, s2b))

    def issue_pk(c, pk, ae, pks, aes):
        pltpu.async_copy(pk_hbm.at[cbase + c], pk, pks)
        pltpu.async_copy(ae_hbm.at[cbase + c], ae, aes)

    issue_pk(0, pk0, ae0, pks0, aes0)
    issue_pk(1, pk1, ae1, pks1, aes1)

    def process(c, b, refill, first):
        pk, ae, sd, rows, exc, pks, aes, gs, s1, s2 = bufs[b]
        pltpu.make_async_copy(pk_hbm.at[0], pk, pks).wait()
        if not first:
            pltpu.make_async_copy(rows, sh_hout.at[sd.at[1]], s1).wait()
            pltpu.make_async_copy(exc, sh_dentot.at[sd.at[1]], s2).wait()

        def unpack(j, cc):
            v = pk[pl.ds(j * 16, 16)]
            sd[0, pl.ds(j * 16, 16)] = jnp.bitwise_and(v, 16383)
            sd[1, pl.ds(j * 16, 16)] = lax.shift_right_logical(v, 14)
            return cc
        lax.fori_loop(0, _CH // 16, unpack, 0)
        gat = pltpu.async_copy(xl_hbm.at[sd.at[0]], rows, gs)
        pltpu.make_async_copy(ae_hbm.at[0], ae, aes).wait()

        def att(j, cc):
            s16 = sd[0, pl.ds(j * 16, 16)]
            d16 = sd[1, pl.ds(j * 16, 16)]
            ae16 = ae[pl.ds(j * 16, 16)]
            al = (plsc.load_gather(asrc_v, [s16])
                  + plsc.load_gather(adst_v, [d16]) + ae16)
            al = jnp.maximum(al, 0.2 * al)
            exc[pl.ds(j * 16, 16)] = jnp.exp(al - mh)
            return cc
        lax.fori_loop(0, _CH // 16, att, 0)
        gat.wait()

        def scale8(i, cc):
            base = i * 8
            ebs = [plsc.load_gather(exc, [jnp.full((16,), base + q, jnp.int32)])
                   for q in range(8)]
            for q in range(8):
                for k in range(HID // 16):
                    rows[base + q, pl.ds(k * 16, 16)] = (
                        rows[base + q, pl.ds(k * 16, 16)] * ebs[q])
            return cc
        lax.fori_loop(0, _CH // 8, scale8, 0)
        pltpu.async_copy(rows, sh_hout.at[sd.at[1]], s1, add=True)
        pltpu.async_copy(exc, sh_dentot.at[sd.at[1]], s2, add=True)
        if refill:
            @pl.when(c + 2 < _NCH)
            def _():
                issue_pk(c + 2, pk, ae, pks, aes)

    process(0, 0, True, True)
    process(1, 1, True, True)

    def pair(cp, carry):
        process(2 * cp, 0, True, False)
        process(2 * cp + 1, 1, True, False)
        return carry
    lax.fori_loop(1, _NCH // 2, pair, 0)
    process(_NCH - 1, 0, False, False)

    pltpu.make_async_copy(rows0, sh_hout.at[sd0.at[1]], s1a).wait()
    pltpu.make_async_copy(exc0, sh_dentot.at[sd0.at[1]], s2a).wait()
    pltpu.make_async_copy(rows1, sh_hout.at[sd1.at[1]], s1b).wait()
    pltpu.make_async_copy(exc1, sh_dentot.at[sd1.at[1]], s2b).wait()

    plsc.subcore_barrier()
    pltpu.sync_copy(sh_hout.at[pl.ds(sid * _ROWS_PT, _ROWS_PT)],
                    outh_hbm.at[cid, pl.ds(sid * _ROWS_PT, _ROWS_PT)])
    pltpu.sync_copy(sh_dentot.at[pl.ds(sid * _ROWS_PT, _ROWS_PT)],
                    outd_hbm.at[cid, pl.ds(sid * _ROWS_PT, _ROWS_PT)])


def _message_passing(pk, ae2d, asrc, adst, mh16, xl):
    mesh = plsc.VectorSubcoreMesh(core_axis_name="c", subcore_axis_name="s")
    f32, i32 = jnp.float32, jnp.int32
    k = pl.kernel(
        _mp_body,
        out_type=[
            jax.ShapeDtypeStruct((2, N_PAD, HID), f32),
            jax.ShapeDtypeStruct((2, N_PAD), f32),
        ],
        mesh=mesh,
        compiler_params=pltpu.CompilerParams(needs_layout_passes=False),
        scratch_types=[
            pltpu.VMEM((N_NODES,), f32),
            pltpu.VMEM((N_NODES,), f32),
            pltpu.VMEM((16,), f32),
            pltpu.VMEM((_ROWS_PT,), f32),
            pltpu.VMEM((_CH,), i32),
            pltpu.VMEM((_CH,), i32),
            pltpu.VMEM((_CH,), f32),
            pltpu.VMEM((_CH,), f32),
            pltpu.VMEM((2, _CH), i32),
            pltpu.VMEM((2, _CH), i32),
            pltpu.VMEM((_CH, HID), f32),
            pltpu.VMEM((_CH, HID), f32),
            pltpu.VMEM((_CH,), f32),
            pltpu.VMEM((_CH,), f32),
            pltpu.VMEM_SHARED((N_PAD, HID), f32),
            pltpu.VMEM_SHARED((N_PAD,), f32),
            pltpu.SemaphoreType.DMA,
            pltpu.SemaphoreType.DMA,
            pltpu.SemaphoreType.DMA,
            pltpu.SemaphoreType.DMA,
            pltpu.SemaphoreType.DMA,
            pltpu.SemaphoreType.DMA,
            pltpu.SemaphoreType.DMA,
            pltpu.SemaphoreType.DMA,
            pltpu.SemaphoreType.DMA,
            pltpu.SemaphoreType.DMA,
        ],
    )
    return k(pk, ae2d, asrc, adst, mh16, xl)


def _finish_body(p0_ref, p1_ref, d0_ref, d1_ref, bg_ref, n2g_ref,
                 h0_ref, p_ref):
    den = d0_ref[...] + d1_ref[...] + 1e-16
    hout = (p0_ref[...] + p1_ref[...]) / den + bg_ref[...]
    n = hout.shape[0]
    bs = p_ref.shape[0]
    gi = lax.broadcasted_iota(jnp.int32, (n, bs), 1)
    oh = (n2g_ref[...] == gi).astype(jnp.float32)
    dn = (((0,), (0,)), ((), ()))
    psum = lax.dot_general(oh, hout, dn, preferred_element_type=jnp.float32)
    cnt = lax.dot_general(oh, jnp.ones((n, 1), jnp.float32), dn,
                          preferred_element_type=jnp.float32)
    p_ref[...] = jax.nn.relu(psum / jnp.maximum(cnt, 1.0))
    ri = lax.broadcasted_iota(jnp.int32, (n, bs), 0)
    oh0 = (ri == gi * (n // bs)).astype(jnp.float32)
    h0_ref[...] = lax.dot_general(oh0, hout, dn,
                                  preferred_element_type=jnp.float32)


def _finish(part0, part1, d0, d1, bg, n2g):
    return pl.pallas_call(
        _finish_body,
        out_shape=[
            jax.ShapeDtypeStruct((BS, HID), jnp.float32),
            jax.ShapeDtypeStruct((BS, HID), jnp.float32),
        ],
    )(part0, part1, d0, d1, bg, n2g)


def kernel(qa_emb, x, node_ids, node_types, node_scores, edge_index,
           edge_type, edge_attr, node2graph, W_nt, b_nt, W_x2h, b_x2h, W_e1,
           b_e1, W_e2, b_e2, W_gat, att_src, att_dst, W_edge, att_edge,
           b_gat):
    xl, a_s, a_d, amax = _node_enc(
        x, qa_emb, node_types, node_scores, W_nt, b_nt.reshape(1, -1),
        W_x2h, b_x2h.reshape(1, -1), W_gat,
        att_src.reshape(-1, 1), att_dst.reshape(-1, 1))
    ae, aemax = _edge_enc(edge_attr, W_e1, b_e1.reshape(1, -1), W_e2,
                          b_e2.reshape(1, -1), W_edge,
                          att_edge.reshape(-1, 1))
    msum = amax[0, 0] + amax[0, 1] + jnp.max(aemax)
    mh = jnp.maximum(msum, 0.2 * msum)
    mh16 = jnp.broadcast_to(mh, (16,))
    src = edge_index[0].astype(jnp.int32)
    dst = edge_index[1].astype(jnp.int32)
    pk = (src | (dst << 14)).reshape(N_EDGES // _CH, _CH)
    ae2d = ae.reshape(N_EDGES // _CH, _CH)
    hpart, dpart = _message_passing(pk, ae2d, a_s.reshape(-1),
                                    a_d.reshape(-1), mh16, xl)
    h0, p = _finish(hpart[0, :N_NODES], hpart[1, :N_NODES],
                    dpart[0, :N_NODES].reshape(-1, 1),
                    dpart[1, :N_NODES].reshape(-1, 1),
                    b_gat.reshape(1, -1), node2graph.reshape(-1, 1))
    return (h0, p)
